# Initial kernel scaffold; baseline (speedup 1.0000x reference)
import jax
import jax.numpy as jnp
from jax.experimental import pallas as pl


def kernel(input_ids, emb, pos_emb, W_out):
    raise NotImplementedError("write your pallas kernel here")


# speedup vs baseline: 24.5117x; 24.5117x over previous
import jax
import jax.numpy as jnp
from jax.experimental import pallas as pl
from jax.experimental.pallas import tpu as pltpu

VPAD = 128


def _fused_kernel(tok_ref, emb_ref, pos_ref, w_ref, out_ref):
    tok = tok_ref[...]
    B, T = tok.shape
    non_pad = (tok != 0).astype(jnp.int32)
    t_iota = jax.lax.broadcasted_iota(jnp.int32, (B, T), 1)

    prev = pltpu.roll(non_pad, 1, 1)
    prev = jnp.where(t_iota == 0, 0, prev)
    starts = (non_pad == 1) & (prev == 0)

    BIG = T + 1
    t_or_big = jnp.where(starts, t_iota, BIG)
    s1 = jnp.min(t_or_big, axis=1, keepdims=True)
    t2 = jnp.where(starts & (t_iota > s1), t_iota, BIG)
    s2 = jnp.min(t2, axis=1, keepdims=True)
    s = jnp.where(s2 >= BIG, 0, s2)

    lengths = jnp.sum(non_pad * (t_iota >= s).astype(jnp.int32), axis=1)
    L = jnp.max(lengths)

    endw = s + L
    base_w = ((t_iota >= s) & (t_iota < jnp.minimum(endw, T))).astype(jnp.int32)
    extra = jnp.maximum(endw - T, 0)
    w_int = base_w + jnp.where(t_iota == T - 1, extra, 0)
    ww_i = w_int * non_pad
    cnt_i = jnp.sum(ww_i, axis=1, keepdims=True)
    ww = ww_i.astype(jnp.float32)

    v_iota = jax.lax.broadcasted_iota(jnp.int32, (1, 1, VPAD), 2)
    counts = jnp.zeros((B, VPAD), jnp.float32)
    CH = 512
    for c in range(T // CH):
        tok_c = tok[:, c * CH:(c + 1) * CH]
        ww_c = ww[:, c * CH:(c + 1) * CH]
        oh = (tok_c[:, :, None] == v_iota).astype(jnp.float32)
        counts = counts + jnp.sum(oh * ww_c[:, :, None], axis=1)

    pooled_emb = jnp.dot(counts, emb_ref[...], preferred_element_type=jnp.float32)
    posmask = (t_iota < cnt_i).astype(jnp.float32)
    pooled_pos = jnp.dot(posmask, pos_ref[...], preferred_element_type=jnp.float32)
    pooled = (pooled_emb + pooled_pos) / cnt_i.astype(jnp.float32)
    out_ref[...] = jnp.dot(pooled, w_ref[...], preferred_element_type=jnp.float32)


def kernel(input_ids, emb, pos_emb, W_out):
    B, T = input_ids.shape
    V, D = emb.shape
    emb_p = jnp.zeros((VPAD, D), emb.dtype).at[:V, :].set(emb)
    return pl.pallas_call(
        _fused_kernel,
        out_shape=jax.ShapeDtypeStruct((B, D), jnp.float32),
    )(input_ids, emb_p, pos_emb, W_out)
